# baseline (speedup 1.0000x reference)
import functools

import jax
import jax.numpy as jnp
from jax import lax
from jax.experimental import pallas as pl
from jax.experimental.pallas import tpu as pltpu
from jax.experimental.pallas import tpu_sc as plsc

N_NODES = 10000
D = 128
N_EDGES = 320000

NC = 2
NS = 16
NW = NC * NS

B = 128
DW = 128
NB_W = 80
E_W = NB_W * B
E_PAD = NW * E_W
N_PAD = 10240
ROWS_W = N_PAD // NS

_mesh = plsc.VectorSubcoreMesh(core_axis_name="c", subcore_axis_name="s")


@functools.partial(
    pl.kernel,
    out_type=jax.ShapeDtypeStruct((NC, N_PAD, DW), jnp.float32),
    mesh=_mesh,
    scratch_types=[
        pltpu.VMEM((NB_W, B), jnp.int32),
        pltpu.VMEM((B, DW), jnp.float32),
        pltpu.VMEM_SHARED((N_PAD, DW), jnp.float32),
        [pltpu.SemaphoreType.DMA for _ in range(4)],
    ],
)
def _sc_degree(dst_hbm, ones_hbm, zrow_hbm, deg_out, dst_v, ones_v, deg_sh,
               dsems):
    core = lax.axis_index("c")
    sid = lax.axis_index("s")
    wid = core * NS + sid
    pltpu.sync_copy(zrow_hbm, deg_sh.at[pl.ds(sid * ROWS_W, ROWS_W), :])
    pltpu.sync_copy(ones_hbm, ones_v)
    pltpu.sync_copy(dst_hbm.at[pl.ds(wid * NB_W, NB_W), :], dst_v)
    plsc.subcore_barrier()

    def outer(o, carry):
        for j in range(4):
            b = o * 4 + j

            @pl.when(o > 0)
            def _():
                pltpu.make_async_copy(ones_v, deg_sh.at[dst_v.at[b - 4]],
                                      dsems[j]).wait()

            pltpu.async_copy(ones_v, deg_sh.at[dst_v.at[b]], dsems[j],
                             add=True)
        return carry

    lax.fori_loop(0, NB_W // 4, outer, 0)
    for j in range(4):
        pltpu.make_async_copy(ones_v, deg_sh.at[dst_v.at[NB_W - 4 + j]],
                              dsems[j]).wait()
    plsc.subcore_barrier()
    pltpu.sync_copy(deg_sh.at[pl.ds(sid * ROWS_W, ROWS_W), :],
                    deg_out.at[core, pl.ds(sid * ROWS_W, ROWS_W), :])


NBUF = 2
CH = 4
NPAIR = NB_W // (2 * CH)
SPLIT = 4
RG = B // SPLIT
CROWS = CH * SPLIT


@functools.partial(
    pl.kernel,
    out_type=jax.ShapeDtypeStruct((NC, N_PAD, D), jnp.float32),
    mesh=_mesh,
    scratch_types=[
        pltpu.VMEM((NB_W, B), jnp.int32),
        [pltpu.VMEM((CROWS, RG), jnp.int32) for _ in range(2)],
        [pltpu.VMEM((B, D), jnp.float32) for _ in range(NBUF)],
        pltpu.VMEM_SHARED((N_PAD, D), jnp.float32),
        [pltpu.SemaphoreType.DMA for _ in range(NBUF * SPLIT)],
        [pltpu.SemaphoreType.DMA for _ in range(2)],
    ],
)
def _sc_aggregate(h_hbm, src_hbm, dst_hbm, zrow_hbm, parts_out,
                  dst_v, schunks, bufs, agg_sh, gsems, csems):
    core = lax.axis_index("c")
    sid = lax.axis_index("s")
    wid = core * NS + sid
    base = wid * NB_W
    rbase = base * SPLIT
    pltpu.sync_copy(zrow_hbm, agg_sh.at[pl.ds(sid * ROWS_W, ROWS_W), :])
    pltpu.sync_copy(dst_hbm.at[pl.ds(base, NB_W), :], dst_v)
    pltpu.sync_copy(src_hbm.at[pl.ds(rbase, CROWS), :], schunks[0])
    plsc.subcore_barrier()

    def gathers(k, rr, j):
        for s in range(SPLIT):
            pltpu.async_copy(h_hbm.at[schunks[k].at[SPLIT * rr + s]],
                             bufs[j].at[pl.ds(RG * s, RG), :],
                             gsems[j * SPLIT + s])

    def wait_gathers(j):
        for s in range(SPLIT):
            pltpu.make_async_copy(h_hbm.at[schunks[0].at[0]],
                                  bufs[j].at[pl.ds(RG * s, RG), :],
                                  gsems[j * SPLIT + s]).wait()

    for j in range(NBUF):
        gathers(0, j, j)

    def pair(o, carry):
        b0 = o * 2 * CH
        pltpu.async_copy(
            src_hbm.at[pl.ds(rbase + (b0 + CH) * SPLIT, CROWS), :],
            schunks[1], csems[1])
        for j in range(2 * CH):
            b = b0 + j
            r = j + NBUF
            if r == CH:
                pltpu.make_async_copy(
                    src_hbm.at[pl.ds(rbase + (b0 + CH) * SPLIT, CROWS), :],
                    schunks[1], csems[1]).wait()
            if j == CH:
                @pl.when(b0 + 2 * CH < NB_W)
                def _():
                    pltpu.async_copy(
                        src_hbm.at[pl.ds(rbase + (b0 + 2 * CH) * SPLIT,
                                         CROWS), :],
                        schunks[0], csems[0])
            if r == 2 * CH:
                @pl.when(b0 + 2 * CH < NB_W)
                def _():
                    pltpu.make_async_copy(
                        src_hbm.at[pl.ds(rbase + (b0 + 2 * CH) * SPLIT,
                                         CROWS), :],
                        schunks[0], csems[0]).wait()
            wait_gathers(j % NBUF)
            pltpu.sync_copy(bufs[j % NBUF], agg_sh.at[dst_v.at[b]], add=True)

            @pl.when(b + NBUF < NB_W)
            def _():
                if r < CH:
                    gathers(0, r, j % NBUF)
                elif r < 2 * CH:
                    gathers(1, r - CH, j % NBUF)
                else:
                    gathers(0, r - 2 * CH, j % NBUF)
        return carry

    lax.fori_loop(0, NPAIR, pair, 0)
    plsc.subcore_barrier()
    pltpu.sync_copy(agg_sh.at[pl.ds(sid * ROWS_W, ROWS_W), :],
                    parts_out.at[core, pl.ds(sid * ROWS_W, ROWS_W), :])


_TC_ROWS = 400


def _scale_from_deg(d_ref):
    deg = d_ref[0, :, 0:1] + d_ref[1, :, 0:1]
    return jnp.where(deg > 0.0, lax.rsqrt(deg), 0.0)


def _tc_scale_body(f_ref, d_ref, o_ref):
    o_ref[...] = f_ref[...] * _scale_from_deg(d_ref)


def _tc_combine_body(p_ref, d_ref, o_ref):
    o_ref[...] = (p_ref[0] + p_ref[1]) * _scale_from_deg(d_ref)


_deg_spec = pl.BlockSpec((NC, _TC_ROWS, DW), lambda i: (0, i, 0))

_tc_scale = pl.pallas_call(
    _tc_scale_body,
    grid=(N_NODES // _TC_ROWS,),
    in_specs=[pl.BlockSpec((_TC_ROWS, D), lambda i: (i, 0)), _deg_spec],
    out_specs=pl.BlockSpec((_TC_ROWS, D), lambda i: (i, 0)),
    out_shape=jax.ShapeDtypeStruct((N_NODES, D), jnp.float32),
)

_tc_combine = pl.pallas_call(
    _tc_combine_body,
    grid=(N_NODES // _TC_ROWS,),
    in_specs=[pl.BlockSpec((NC, _TC_ROWS, D), lambda i: (0, i, 0)), _deg_spec],
    out_specs=pl.BlockSpec((_TC_ROWS, D), lambda i: (i, 0)),
    out_shape=jax.ShapeDtypeStruct((N_NODES, D), jnp.float32),
)


def kernel(features, edge_index):
    src = edge_index[0]
    dst = edge_index[1]
    n_extra = E_PAD - N_EDGES
    src2 = jnp.concatenate(
        [src, jnp.zeros((n_extra,), jnp.int32)]).reshape(NW * NB_W * SPLIT, RG)
    dst2 = jnp.concatenate(
        [dst, jnp.full((n_extra,), N_NODES, jnp.int32)]).reshape(NW * NB_W, B)
    ones8 = jnp.ones((B, DW), jnp.float32)
    z8 = jnp.zeros((ROWS_W, DW), jnp.float32)
    z128 = jnp.zeros((ROWS_W, D), jnp.float32)

    deg = _sc_degree(dst2, ones8, z8)
    h = _tc_scale(features, deg)
    parts = _sc_aggregate(h, src2, dst2, z128)
    return _tc_combine(parts, deg)


# speedup vs baseline: 1.0581x; 1.0581x over previous
import functools

import jax
import jax.numpy as jnp
from jax import lax
from jax.experimental import pallas as pl
from jax.experimental.pallas import tpu as pltpu
from jax.experimental.pallas import tpu_sc as plsc

N_NODES = 10000
D = 128
N_EDGES = 320000

NC = 2
NS = 16
NW = NC * NS

B = 128
DW = 128
NB_W = 80
E_W = NB_W * B
E_PAD = NW * E_W
N_PAD = 10240
ROWS_W = N_PAD // NS

_mesh = plsc.VectorSubcoreMesh(core_axis_name="c", subcore_axis_name="s")


@functools.partial(
    pl.kernel,
    out_type=jax.ShapeDtypeStruct((NC, N_PAD, DW), jnp.float32),
    mesh=_mesh,
    scratch_types=[
        pltpu.VMEM((NB_W, B), jnp.int32),
        pltpu.VMEM((B, DW), jnp.float32),
        pltpu.VMEM_SHARED((N_PAD, DW), jnp.float32),
        [pltpu.SemaphoreType.DMA for _ in range(4)],
    ],
)
def _sc_degree(dst_hbm, ones_hbm, zrow_hbm, deg_out, dst_v, ones_v, deg_sh,
               dsems):
    core = lax.axis_index("c")
    sid = lax.axis_index("s")
    wid = core * NS + sid
    pltpu.sync_copy(zrow_hbm, deg_sh.at[pl.ds(sid * ROWS_W, ROWS_W), :])
    pltpu.sync_copy(ones_hbm, ones_v)
    pltpu.sync_copy(dst_hbm.at[pl.ds(wid * NB_W, NB_W), :], dst_v)
    plsc.subcore_barrier()

    def outer(o, carry):
        for j in range(4):
            b = o * 4 + j

            @pl.when(o > 0)
            def _():
                pltpu.make_async_copy(ones_v, deg_sh.at[dst_v.at[b - 4]],
                                      dsems[j]).wait()

            pltpu.async_copy(ones_v, deg_sh.at[dst_v.at[b]], dsems[j],
                             add=True)
        return carry

    lax.fori_loop(0, NB_W // 4, outer, 0)
    for j in range(4):
        pltpu.make_async_copy(ones_v, deg_sh.at[dst_v.at[NB_W - 4 + j]],
                              dsems[j]).wait()
    plsc.subcore_barrier()
    pltpu.sync_copy(deg_sh.at[pl.ds(sid * ROWS_W, ROWS_W), :],
                    deg_out.at[core, pl.ds(sid * ROWS_W, ROWS_W), :])


NBUF = 2
NB_FAST = 128
NB_SLOW = 2 * NB_W - NB_FAST
CH = 4
NPAIR = NB_W // (2 * CH)
SPLIT = 4
RG = B // SPLIT
CROWS = CH * SPLIT


@functools.partial(
    pl.kernel,
    out_type=jax.ShapeDtypeStruct((NC, N_PAD, D), jnp.float32),
    mesh=_mesh,
    scratch_types=[
        pltpu.VMEM((NB_FAST // 2, B), jnp.int32),
        [pltpu.VMEM((CROWS, RG), jnp.int32) for _ in range(2)],
        [pltpu.VMEM((B, D), jnp.float32) for _ in range(NBUF)],
        pltpu.VMEM_SHARED((N_PAD, D), jnp.float32),
        [pltpu.SemaphoreType.DMA for _ in range(NBUF * SPLIT)],
        [pltpu.SemaphoreType.DMA for _ in range(2)],
    ],
)
def _sc_aggregate(h_hbm, src_hbm, dst_hbm, zrow_hbm, parts_out,
                  dst_v, schunks, bufs, agg_sh, gsems, csems):
    core = lax.axis_index("c")
    sid = lax.axis_index("s")
    nb_w = jnp.where(core == 0, NB_FAST, NB_SLOW)
    base = jnp.where(core == 0, sid * NB_FAST,
                     NS * NB_FAST + sid * NB_SLOW)
    rbase = base * SPLIT
    pltpu.sync_copy(zrow_hbm, agg_sh.at[pl.ds(sid * ROWS_W, ROWS_W), :])
    pltpu.sync_copy(dst_hbm.at[pl.ds(base, NB_FAST // 2), :], dst_v)
    pltpu.sync_copy(src_hbm.at[pl.ds(rbase, CROWS), :], schunks[0])
    plsc.subcore_barrier()

    def gathers(k, rr, j):
        for s in range(SPLIT):
            pltpu.async_copy(h_hbm.at[schunks[k].at[SPLIT * rr + s]],
                             bufs[j].at[pl.ds(RG * s, RG), :],
                             gsems[j * SPLIT + s])

    def wait_gathers(j):
        for s in range(SPLIT):
            pltpu.make_async_copy(h_hbm.at[schunks[0].at[0]],
                                  bufs[j].at[pl.ds(RG * s, RG), :],
                                  gsems[j * SPLIT + s]).wait()

    for j in range(NBUF):
        gathers(0, j, j)

    def pair(o, carry):
        b0 = o * 2 * CH

        @pl.when(b0 == NB_FAST // 2)
        def _():
            pltpu.sync_copy(
                dst_hbm.at[pl.ds(base + NB_FAST // 2, NB_FAST // 2), :],
                dst_v)

        pltpu.async_copy(
            src_hbm.at[pl.ds(rbase + (b0 + CH) * SPLIT, CROWS), :],
            schunks[1], csems[1])
        for j in range(2 * CH):
            b = b0 + j
            r = j + NBUF
            if r == CH:
                pltpu.make_async_copy(
                    src_hbm.at[pl.ds(rbase + (b0 + CH) * SPLIT, CROWS), :],
                    schunks[1], csems[1]).wait()
            if j == CH:
                @pl.when(b0 + 2 * CH < nb_w)
                def _():
                    pltpu.async_copy(
                        src_hbm.at[pl.ds(rbase + (b0 + 2 * CH) * SPLIT,
                                         CROWS), :],
                        schunks[0], csems[0])
            if r == 2 * CH:
                @pl.when(b0 + 2 * CH < nb_w)
                def _():
                    pltpu.make_async_copy(
                        src_hbm.at[pl.ds(rbase + (b0 + 2 * CH) * SPLIT,
                                         CROWS), :],
                        schunks[0], csems[0]).wait()
            wait_gathers(j % NBUF)
            pltpu.sync_copy(bufs[j % NBUF],
                            agg_sh.at[dst_v.at[lax.rem(b, NB_FAST // 2)]],
                            add=True)

            @pl.when(b + NBUF < nb_w)
            def _():
                if r < CH:
                    gathers(0, r, j % NBUF)
                elif r < 2 * CH:
                    gathers(1, r - CH, j % NBUF)
                else:
                    gathers(0, r - 2 * CH, j % NBUF)
        return carry

    lax.fori_loop(0, nb_w // (2 * CH), pair, 0)
    plsc.subcore_barrier()
    pltpu.sync_copy(agg_sh.at[pl.ds(sid * ROWS_W, ROWS_W), :],
                    parts_out.at[core, pl.ds(sid * ROWS_W, ROWS_W), :])


_TC_ROWS = 400


def _scale_from_deg(d_ref):
    deg = d_ref[0, :, 0:1] + d_ref[1, :, 0:1]
    return jnp.where(deg > 0.0, lax.rsqrt(deg), 0.0)


def _tc_scale_body(f_ref, d_ref, o_ref):
    o_ref[...] = f_ref[...] * _scale_from_deg(d_ref)


def _tc_combine_body(p_ref, d_ref, o_ref):
    o_ref[...] = (p_ref[0] + p_ref[1]) * _scale_from_deg(d_ref)


_deg_spec = pl.BlockSpec((NC, _TC_ROWS, DW), lambda i: (0, i, 0))

_tc_scale = pl.pallas_call(
    _tc_scale_body,
    grid=(N_NODES // _TC_ROWS,),
    in_specs=[pl.BlockSpec((_TC_ROWS, D), lambda i: (i, 0)), _deg_spec],
    out_specs=pl.BlockSpec((_TC_ROWS, D), lambda i: (i, 0)),
    out_shape=jax.ShapeDtypeStruct((N_NODES, D), jnp.float32),
)

_tc_combine = pl.pallas_call(
    _tc_combine_body,
    grid=(N_NODES // _TC_ROWS,),
    in_specs=[pl.BlockSpec((NC, _TC_ROWS, D), lambda i: (0, i, 0)), _deg_spec],
    out_specs=pl.BlockSpec((_TC_ROWS, D), lambda i: (i, 0)),
    out_shape=jax.ShapeDtypeStruct((N_NODES, D), jnp.float32),
)


def kernel(features, edge_index):
    src = edge_index[0]
    dst = edge_index[1]
    n_extra = E_PAD - N_EDGES
    src2 = jnp.concatenate(
        [src, jnp.zeros((n_extra,), jnp.int32)]).reshape(NW * NB_W * SPLIT, RG)
    dst2 = jnp.concatenate(
        [dst, jnp.full((n_extra + (NB_FAST // 2) * B,), N_NODES, jnp.int32)]
    ).reshape(NW * NB_W + NB_FAST // 2, B)
    ones8 = jnp.ones((B, DW), jnp.float32)
    z8 = jnp.zeros((ROWS_W, DW), jnp.float32)
    z128 = jnp.zeros((ROWS_W, D), jnp.float32)

    deg = _sc_degree(dst2, ones8, z8)
    h = _tc_scale(features, deg)
    parts = _sc_aggregate(h, src2, dst2, z128)
    return _tc_combine(parts, deg)
